# baseline (speedup 1.0000x reference)
import functools

import jax
import jax.numpy as jnp
from jax import lax
from jax.experimental import pallas as pl
from jax.experimental.pallas import tpu as pltpu
from jax.experimental.pallas import tpu_sc as plsc

N_NODES = 10000
N_EDGES = 320000
D = 128

NC = 2
NS = 16
NW = NC * NS
L = 16

NP = 10240
ROWS_PER_TILE = NP // NW
ROWS_PER_SUB = NP // NS

CHUNK = 128
CPT = 80
EP = NW * CPT * CHUNK
NROWS_E = EP // CHUNK
CPS = NROWS_E // NS
GRP = 16


def _fill_const(ref, n, value):
    v = jnp.full((L,), value, dtype=jnp.float32)
    for j in range(n // L):
        ref[pl.ds(j * L, L)] = v


def _newton_rsqrt(x):
    y = 1.0 / x
    for _ in range(30):
        y = y * (1.5 - 0.5 * x * y * y)
    return y


def _zero_acc(acc_sp, rows0, s):
    def zrow(r, carry):
        z = jnp.zeros((L,), jnp.float32)
        for j in range(D // L):
            rows0[r, pl.ds(j * L, L)] = z
        return carry

    lax.fori_loop(0, CHUNK, zrow, 0)
    for t in range(ROWS_PER_SUB // CHUNK):
        pltpu.sync_copy(
            rows0, acc_sp.at[pl.ds(s * ROWS_PER_SUB + t * CHUNK, CHUNK)])


def _hop_stream(g_hbm, src_hbm, dst_hbm, out_hbm,
                acc_sp, sidx, didx, rows0, rows1, sem0, sem1, c, s, w):
    for g in range(CPT // GRP):
        gbase = w * CPT + g * GRP
        pltpu.sync_copy(src_hbm.at[pl.ds(gbase, GRP)], sidx)
        pltpu.sync_copy(dst_hbm.at[pl.ds(gbase, GRP)], didx)
        pltpu.async_copy(g_hbm.at[sidx.at[0]], rows0, sem0)

        def chunk_body(k, carry):
            b0 = 2 * k
            b1 = 2 * k + 1
            pltpu.make_async_copy(g_hbm.at[sidx.at[b0]], rows0, sem0).wait()
            pltpu.async_copy(g_hbm.at[sidx.at[b1]], rows1, sem1)
            pltpu.sync_copy(rows0, acc_sp.at[didx.at[b0]], add=True)
            nxt = jnp.minimum(b0 + 2, GRP - 1)
            pltpu.make_async_copy(g_hbm.at[sidx.at[b1]], rows1, sem1).wait()
            pltpu.async_copy(g_hbm.at[sidx.at[nxt]], rows0, sem0)
            pltpu.sync_copy(rows1, acc_sp.at[didx.at[b1]], add=True)
            return carry

        lax.fori_loop(0, GRP // 2, chunk_body, 0)
        pltpu.make_async_copy(g_hbm.at[sidx.at[GRP - 1]], rows0, sem0).wait()
    plsc.subcore_barrier()

    pltpu.sync_copy(acc_sp.at[pl.ds(s * ROWS_PER_SUB, ROWS_PER_SUB)],
                    out_hbm.at[c, pl.ds(s * ROWS_PER_SUB, ROWS_PER_SUB)])


def _hop1_body(feat_hbm, src_hbm, dst_hbm,
               norm_hbm, invd_hbm, g0_hbm, p_hbm,
               acc_sp, deg_sp, sidx, didx, rows0, rows1,
               buf_v, ones_v, nrm_v, ivd_v, sem0, sem1):
    c = lax.axis_index("c")
    s = lax.axis_index("s")
    w = s * NC + c

    _fill_const(buf_v, ROWS_PER_SUB, 0.0)
    _fill_const(ones_v, CHUNK, 1.0)
    pltpu.sync_copy(buf_v, deg_sp.at[pl.ds(s * ROWS_PER_SUB, ROWS_PER_SUB)])
    _zero_acc(acc_sp, rows0, s)
    plsc.subcore_barrier()

    for g in range(CPS // GRP):
        pltpu.sync_copy(dst_hbm.at[pl.ds(s * CPS + g * GRP, GRP)], didx)

        def dchunk_body(k, carry):
            pltpu.sync_copy(ones_v, deg_sp.at[didx.at[k]], add=True)
            return carry

        lax.fori_loop(0, GRP, dchunk_body, 0)
    plsc.subcore_barrier()

    base = s * ROWS_PER_SUB
    pltpu.sync_copy(deg_sp.at[pl.ds(base, ROWS_PER_SUB)], buf_v)
    for j in range(ROWS_PER_SUB // L):
        d = jnp.maximum(buf_v[pl.ds(j * L, L)], 1.0)
        y = _newton_rsqrt(d)
        nrm_v[pl.ds(j * L, L)] = y
        ivd_v[pl.ds(j * L, L)] = 1.0 / d
    pltpu.sync_copy(nrm_v, norm_hbm.at[pl.ds(base, ROWS_PER_SUB)])
    pltpu.sync_copy(ivd_v, invd_hbm.at[pl.ds(base, ROWS_PER_SUB)])

    for t in range(ROWS_PER_SUB // CHUNK):
        rbase = base + t * CHUNK
        pltpu.sync_copy(feat_hbm.at[pl.ds(rbase, CHUNK)], rows0)

        def sblk_body(g, carry, t=t):
            nrm16 = nrm_v[pl.ds(t * CHUNK + g * L, L)]
            base_r = g * L
            for u in range(L):
                nv = jax.lax.broadcast(nrm16[u], (L,))
                for j in range(D // L):
                    rows0[base_r + u, pl.ds(j * L, L)] = (
                        rows0[base_r + u, pl.ds(j * L, L)] * nv)
            return carry

        lax.fori_loop(0, CHUNK // L, sblk_body, 0)
        pltpu.sync_copy(rows0, g0_hbm.at[pl.ds(rbase, CHUNK)])
    plsc.subcore_barrier()

    _hop_stream(g0_hbm, src_hbm, dst_hbm, p_hbm,
                acc_sp, sidx, didx, rows0, rows1, sem0, sem1, c, s, w)


def _hop2_body(p_hbm, invd_hbm, src_hbm, dst_hbm, g1_hbm, q_hbm,
               acc_sp, sidx, didx, rows0, rows1, invd_v, sem0, sem1):
    c = lax.axis_index("c")
    s = lax.axis_index("s")
    w = s * NC + c
    _zero_acc(acc_sp, rows0, s)

    base = s * ROWS_PER_SUB
    pltpu.sync_copy(invd_hbm.at[pl.ds(base, ROWS_PER_SUB)], invd_v)
    for t in range(ROWS_PER_SUB // CHUNK):
        rbase = base + t * CHUNK
        pltpu.sync_copy(p_hbm.at[0, pl.ds(rbase, CHUNK)], rows0)
        pltpu.sync_copy(p_hbm.at[1, pl.ds(rbase, CHUNK)], rows1)

        def blk_body(g, carry, t=t):
            iv16 = invd_v[pl.ds(t * CHUNK + g * L, L)]
            base_r = g * L
            for u in range(L):
                iv = jax.lax.broadcast(iv16[u], (L,))
                for j in range(D // L):
                    rows0[base_r + u, pl.ds(j * L, L)] = (
                        rows0[base_r + u, pl.ds(j * L, L)]
                        + rows1[base_r + u, pl.ds(j * L, L)]) * iv
            return carry

        lax.fori_loop(0, CHUNK // L, blk_body, 0)
        pltpu.sync_copy(rows0, g1_hbm.at[pl.ds(rbase, CHUNK)])
    plsc.subcore_barrier()

    _hop_stream(g1_hbm, src_hbm, dst_hbm, q_hbm,
                acc_sp, sidx, didx, rows0, rows1, sem0, sem1, c, s, w)


_hop1 = functools.partial(
    pl.kernel,
    out_type=(
        jax.ShapeDtypeStruct((NP,), jnp.float32),
        jax.ShapeDtypeStruct((NP,), jnp.float32),
        jax.ShapeDtypeStruct((NP, D), jnp.float32),
        jax.ShapeDtypeStruct((NC, NP, D), jnp.float32),
    ),
    mesh=plsc.VectorSubcoreMesh(core_axis_name="c", subcore_axis_name="s"),
    scratch_types=[
        pltpu.VMEM_SHARED((NP, D), jnp.float32),
        pltpu.VMEM_SHARED((NP,), jnp.float32),
        pltpu.VMEM((GRP, CHUNK), jnp.int32),
        pltpu.VMEM((GRP, CHUNK), jnp.int32),
        pltpu.VMEM((CHUNK, D), jnp.float32),
        pltpu.VMEM((CHUNK, D), jnp.float32),
        pltpu.VMEM((ROWS_PER_SUB,), jnp.float32),
        pltpu.VMEM((CHUNK,), jnp.float32),
        pltpu.VMEM((ROWS_PER_SUB,), jnp.float32),
        pltpu.VMEM((ROWS_PER_SUB,), jnp.float32),
        pltpu.SemaphoreType.DMA,
        pltpu.SemaphoreType.DMA,
    ],
)(_hop1_body)


_hop2 = functools.partial(
    pl.kernel,
    out_type=(
        jax.ShapeDtypeStruct((NP, D), jnp.float32),
        jax.ShapeDtypeStruct((NC, NP, D), jnp.float32),
    ),
    mesh=plsc.VectorSubcoreMesh(core_axis_name="c", subcore_axis_name="s"),
    scratch_types=[
        pltpu.VMEM_SHARED((NP, D), jnp.float32),
        pltpu.VMEM((GRP, CHUNK), jnp.int32),
        pltpu.VMEM((GRP, CHUNK), jnp.int32),
        pltpu.VMEM((CHUNK, D), jnp.float32),
        pltpu.VMEM((CHUNK, D), jnp.float32),
        pltpu.VMEM((ROWS_PER_SUB,), jnp.float32),
        pltpu.SemaphoreType.DMA,
        pltpu.SemaphoreType.DMA,
    ],
)(_hop2_body)


def _out_body(q_ref, norm_ref, w_ref, b_ref, out_ref):
    h = (q_ref[0] + q_ref[1]) * norm_ref[...]
    h = h[:N_NODES]
    out_ref[...] = (
        jnp.dot(h, w_ref[...], preferred_element_type=jnp.float32)
        + b_ref[...]
    )


def kernel(node_feat, edge_index, W, b):
    src = edge_index[0].astype(jnp.int32)
    dst = edge_index[1].astype(jnp.int32)
    pad_e = EP - N_EDGES
    src = jnp.concatenate([src, jnp.zeros((pad_e,), jnp.int32)])
    dst = jnp.concatenate([dst, jnp.full((pad_e,), NP - 1, jnp.int32)])
    src2d = src.reshape(NROWS_E, CHUNK)
    dst2d = dst.reshape(NROWS_E, CHUNK)
    feat = jnp.pad(node_feat, ((0, NP - N_NODES), (0, 0)))

    norm, invd, _, p = _hop1(feat, src2d, dst2d)

    _, q = _hop2(p, invd, src2d, dst2d)

    out = pl.pallas_call(
        _out_body,
        out_shape=jax.ShapeDtypeStruct((N_NODES, D), jnp.float32),
    )(q, norm.reshape(NP, 1), W, b)
    return out


# speedup vs baseline: 1.0400x; 1.0400x over previous
import functools

import jax
import jax.numpy as jnp
from jax import lax
from jax.experimental import pallas as pl
from jax.experimental.pallas import tpu as pltpu
from jax.experimental.pallas import tpu_sc as plsc

N_NODES = 10000
N_EDGES = 320000
D = 128

NC = 2
NS = 16
NW = NC * NS
L = 16

NP = 10240
ROWS_PER_TILE = NP // NW
ROWS_PER_SUB = NP // NS

CHUNK = 128
CPT = 80
EP = NW * CPT * CHUNK
NROWS_E = EP // CHUNK
CPS = NROWS_E // NS
GRP = 16


def _fill_const(ref, n, value):
    v = jnp.full((L,), value, dtype=jnp.float32)
    for j in range(n // L):
        ref[pl.ds(j * L, L)] = v


def _newton_rsqrt(x):
    y = 1.0 / x
    for _ in range(30):
        y = y * (1.5 - 0.5 * x * y * y)
    return y


def _deg_norm_body(dst_hbm, feat_hbm, norm_hbm, invd_hbm, g0_hbm,
                   deg_sp, didx, buf, ones_v, deg_v, norm_v, invd_v, feat_v):
    c = lax.axis_index("c")
    s = lax.axis_index("s")
    w = s * NC + c

    _fill_const(buf, ROWS_PER_SUB, 0.0)
    _fill_const(ones_v, CHUNK, 1.0)
    pltpu.sync_copy(buf, deg_sp.at[pl.ds(s * ROWS_PER_SUB, ROWS_PER_SUB)])
    plsc.subcore_barrier()

    pltpu.sync_copy(dst_hbm.at[pl.ds(s * CPS, CPS)], didx)

    def chunk_body(k, carry):
        pltpu.sync_copy(ones_v, deg_sp.at[didx.at[k]], add=True)
        return carry

    lax.fori_loop(0, CPS, chunk_body, 0)
    plsc.subcore_barrier()

    base = w * ROWS_PER_TILE
    pltpu.sync_copy(deg_sp.at[pl.ds(base, ROWS_PER_TILE)], deg_v)
    for j in range(ROWS_PER_TILE // L):
        d = jnp.maximum(deg_v[pl.ds(j * L, L)], 1.0)
        y = _newton_rsqrt(d)
        norm_v[pl.ds(j * L, L)] = y
        invd_v[pl.ds(j * L, L)] = 1.0 / d
    pltpu.sync_copy(norm_v, norm_hbm.at[pl.ds(base, ROWS_PER_TILE)])
    pltpu.sync_copy(invd_v, invd_hbm.at[pl.ds(base, ROWS_PER_TILE)])

    pltpu.sync_copy(feat_hbm.at[pl.ds(base, ROWS_PER_TILE)], feat_v)

    def grp_body(g, carry):
        nrm16 = norm_v[pl.ds(g * L, L)]
        base_r = g * L
        for t in range(L):
            nv = jax.lax.broadcast(nrm16[t], (L,))
            for j in range(D // L):
                feat_v[base_r + t, pl.ds(j * L, L)] = (
                    feat_v[base_r + t, pl.ds(j * L, L)] * nv)
        return carry

    lax.fori_loop(0, ROWS_PER_TILE // L, grp_body, 0)
    pltpu.sync_copy(feat_v, g0_hbm.at[pl.ds(base, ROWS_PER_TILE)])


_deg_norm = functools.partial(
    pl.kernel,
    out_type=(
        jax.ShapeDtypeStruct((NP,), jnp.float32),
        jax.ShapeDtypeStruct((NP,), jnp.float32),
        jax.ShapeDtypeStruct((NP, D), jnp.float32),
    ),
    mesh=plsc.VectorSubcoreMesh(core_axis_name="c", subcore_axis_name="s"),
    scratch_types=[
        pltpu.VMEM_SHARED((NP,), jnp.float32),
        pltpu.VMEM((CPS, CHUNK), jnp.int32),
        pltpu.VMEM((ROWS_PER_SUB,), jnp.float32),
        pltpu.VMEM((CHUNK,), jnp.float32),
        pltpu.VMEM((ROWS_PER_TILE,), jnp.float32),
        pltpu.VMEM((ROWS_PER_TILE,), jnp.float32),
        pltpu.VMEM((ROWS_PER_TILE,), jnp.float32),
        pltpu.VMEM((ROWS_PER_TILE, D), jnp.float32),
    ],
)(_deg_norm_body)


def _zero_acc(acc_sp, rows0, s):
    def zrow(r, carry):
        z = jnp.zeros((L,), jnp.float32)
        for j in range(D // L):
            rows0[r, pl.ds(j * L, L)] = z
        return carry

    lax.fori_loop(0, CHUNK, zrow, 0)
    for t in range(ROWS_PER_SUB // CHUNK):
        pltpu.sync_copy(
            rows0, acc_sp.at[pl.ds(s * ROWS_PER_SUB + t * CHUNK, CHUNK)])


def _hop_stream(g_hbm, src_hbm, dst_hbm, out_hbm,
                acc_sp, sidx, didx, rows0, rows1, sem0, sem1, c, s, w):
    for g in range(CPT // GRP):
        gbase = w * CPT + g * GRP
        pltpu.sync_copy(src_hbm.at[pl.ds(gbase, GRP)], sidx)
        pltpu.sync_copy(dst_hbm.at[pl.ds(gbase, GRP)], didx)
        pltpu.async_copy(g_hbm.at[sidx.at[0]], rows0, sem0)

        def chunk_body(k, carry):
            b0 = 2 * k
            b1 = 2 * k + 1
            pltpu.make_async_copy(g_hbm.at[sidx.at[b0]], rows0, sem0).wait()
            pltpu.async_copy(g_hbm.at[sidx.at[b1]], rows1, sem1)
            pltpu.sync_copy(rows0, acc_sp.at[didx.at[b0]], add=True)
            nxt = jnp.minimum(b0 + 2, GRP - 1)
            pltpu.make_async_copy(g_hbm.at[sidx.at[b1]], rows1, sem1).wait()
            pltpu.async_copy(g_hbm.at[sidx.at[nxt]], rows0, sem0)
            pltpu.sync_copy(rows1, acc_sp.at[didx.at[b1]], add=True)
            return carry

        lax.fori_loop(0, GRP // 2, chunk_body, 0)
        pltpu.make_async_copy(g_hbm.at[sidx.at[GRP - 1]], rows0, sem0).wait()
    plsc.subcore_barrier()

    pltpu.sync_copy(acc_sp.at[pl.ds(s * ROWS_PER_SUB, ROWS_PER_SUB)],
                    out_hbm.at[c, pl.ds(s * ROWS_PER_SUB, ROWS_PER_SUB)])


def _hop_body(g_hbm, src_hbm, dst_hbm, p_hbm,
              acc_sp, sidx, didx, rows0, rows1, sem0, sem1):
    c = lax.axis_index("c")
    s = lax.axis_index("s")
    w = s * NC + c
    _zero_acc(acc_sp, rows0, s)
    plsc.subcore_barrier()
    _hop_stream(g_hbm, src_hbm, dst_hbm, p_hbm,
                acc_sp, sidx, didx, rows0, rows1, sem0, sem1, c, s, w)


def _hop2_body(p_hbm, invd_hbm, src_hbm, dst_hbm, g1_hbm, q_hbm,
               acc_sp, sidx, didx, rows0, rows1, invd_v, sem0, sem1):
    c = lax.axis_index("c")
    s = lax.axis_index("s")
    w = s * NC + c
    _zero_acc(acc_sp, rows0, s)

    base = s * ROWS_PER_SUB
    pltpu.sync_copy(invd_hbm.at[pl.ds(base, ROWS_PER_SUB)], invd_v)
    for t in range(ROWS_PER_SUB // CHUNK):
        rbase = base + t * CHUNK
        pltpu.sync_copy(p_hbm.at[0, pl.ds(rbase, CHUNK)], rows0)
        pltpu.sync_copy(p_hbm.at[1, pl.ds(rbase, CHUNK)], rows1)

        def blk_body(g, carry, t=t):
            iv16 = invd_v[pl.ds(t * CHUNK + g * L, L)]
            base_r = g * L
            for u in range(L):
                iv = jax.lax.broadcast(iv16[u], (L,))
                for j in range(D // L):
                    rows0[base_r + u, pl.ds(j * L, L)] = (
                        rows0[base_r + u, pl.ds(j * L, L)]
                        + rows1[base_r + u, pl.ds(j * L, L)]) * iv
            return carry

        lax.fori_loop(0, CHUNK // L, blk_body, 0)
        pltpu.sync_copy(rows0, g1_hbm.at[pl.ds(rbase, CHUNK)])
    plsc.subcore_barrier()

    _hop_stream(g1_hbm, src_hbm, dst_hbm, q_hbm,
                acc_sp, sidx, didx, rows0, rows1, sem0, sem1, c, s, w)


_hop = functools.partial(
    pl.kernel,
    out_type=jax.ShapeDtypeStruct((NC, NP, D), jnp.float32),
    mesh=plsc.VectorSubcoreMesh(core_axis_name="c", subcore_axis_name="s"),
    scratch_types=[
        pltpu.VMEM_SHARED((NP, D), jnp.float32),
        pltpu.VMEM((GRP, CHUNK), jnp.int32),
        pltpu.VMEM((GRP, CHUNK), jnp.int32),
        pltpu.VMEM((CHUNK, D), jnp.float32),
        pltpu.VMEM((CHUNK, D), jnp.float32),
        pltpu.SemaphoreType.DMA,
        pltpu.SemaphoreType.DMA,
    ],
)(_hop_body)


_hop2 = functools.partial(
    pl.kernel,
    out_type=(
        jax.ShapeDtypeStruct((NP, D), jnp.float32),
        jax.ShapeDtypeStruct((NC, NP, D), jnp.float32),
    ),
    mesh=plsc.VectorSubcoreMesh(core_axis_name="c", subcore_axis_name="s"),
    scratch_types=[
        pltpu.VMEM_SHARED((NP, D), jnp.float32),
        pltpu.VMEM((GRP, CHUNK), jnp.int32),
        pltpu.VMEM((GRP, CHUNK), jnp.int32),
        pltpu.VMEM((CHUNK, D), jnp.float32),
        pltpu.VMEM((CHUNK, D), jnp.float32),
        pltpu.VMEM((ROWS_PER_SUB,), jnp.float32),
        pltpu.SemaphoreType.DMA,
        pltpu.SemaphoreType.DMA,
    ],
)(_hop2_body)


def _out_body(q_ref, norm_ref, w_ref, b_ref, out_ref):
    h = (q_ref[0] + q_ref[1]) * norm_ref[...]
    h = h[:N_NODES]
    out_ref[...] = (
        jnp.dot(h, w_ref[...], preferred_element_type=jnp.float32)
        + b_ref[...]
    )


def kernel(node_feat, edge_index, W, b):
    src = edge_index[0].astype(jnp.int32)
    dst = edge_index[1].astype(jnp.int32)
    pad_e = EP - N_EDGES
    src = jnp.concatenate([src, jnp.zeros((pad_e,), jnp.int32)])
    dst = jnp.concatenate([dst, jnp.full((pad_e,), NP - 1, jnp.int32)])
    src2d = src.reshape(NROWS_E, CHUNK)
    dst2d = dst.reshape(NROWS_E, CHUNK)
    feat = jnp.pad(node_feat, ((0, NP - N_NODES), (0, 0)))

    norm, invd, g0 = _deg_norm(dst2d, feat)

    p = _hop(g0, src2d, dst2d)

    _, q = _hop2(p, invd, src2d, dst2d)

    out = pl.pallas_call(
        _out_body,
        out_shape=jax.ShapeDtypeStruct((N_NODES, D), jnp.float32),
    )(q, norm.reshape(NP, 1), W, b)
    return out
